# baseline (speedup 1.0000x reference)
import numpy as np
import jax
import jax.numpy as jnp
from jax import lax
from jax.experimental import pallas as pl

ALPHA = 1.702
LIMIT = 7.0
FT = 512


def _nib2val(n):
    m = n & 7
    mag = jnp.where(
        m == 0, 0.0,
        jnp.where(m == 1, 0.5,
        jnp.where(m == 2, 1.0,
        jnp.where(m == 3, 1.5,
        jnp.where(m == 4, 2.0,
        jnp.where(m == 5, 3.0,
        jnp.where(m == 6, 4.0, 6.0)))))))
    return jnp.where(n >= 8, -mag, mag)


def _dot_nt(a, b):
    return lax.dot_general(a, b, (((1,), (1,)), ((), ())),
                           preferred_element_type=jnp.float32)


def _router_kernel(x_ref, rw_ref, rb_ref, wts_ref):
    x = x_ref[...]
    logits = _dot_nt(x, rw_ref[...]) + rb_ref[...]
    m1 = jnp.max(logits, axis=1, keepdims=True)
    is1 = logits == m1
    masked = jnp.where(is1, -jnp.inf, logits)
    m2 = jnp.max(masked, axis=1, keepdims=True)
    is2 = masked == m2
    p1 = 1.0 / (1.0 + jnp.exp(m2 - m1))
    wts_ref[...] = jnp.where(is1, p1, 0.0) + jnp.where(is2, 1.0 - p1, 0.0)


def _dequant_pair(b_ref, s_ref, sel16):
    s = jnp.dot(s_ref[0].astype(jnp.bfloat16), sel16,
                preferred_element_type=jnp.float32)
    b = b_ref[0].astype(jnp.int32)
    lo = (_nib2val(b & 15) * s).astype(jnp.bfloat16)
    hi = (_nib2val(b >> 4) * s).astype(jnp.bfloat16)
    return lo, hi


def _moe_kernel(xe_ref, xo_ref, wts_ref,
                geb_ref, gob_ref, ueb_ref, uob_ref,
                ges_ref, gos_ref, ues_ref, uos_ref,
                gbe_ref, gbo_ref, ube_ref, ubo_ref,
                dnb_ref, dns_ref, dnbias_ref, out_ref):
    e = pl.program_id(0)
    j = pl.program_id(1)
    nsc = ges_ref.shape[2]

    ci = lax.broadcasted_iota(jnp.int32, (nsc, 16 * nsc), 1) // 16
    bi = lax.broadcasted_iota(jnp.int32, (nsc, 16 * nsc), 0)
    sel16 = (ci == bi).astype(jnp.bfloat16)

    wge_lo, wge_hi = _dequant_pair(geb_ref, ges_ref, sel16)
    wgo_lo, wgo_hi = _dequant_pair(gob_ref, gos_ref, sel16)
    wue_lo, wue_hi = _dequant_pair(ueb_ref, ues_ref, sel16)
    wuo_lo, wuo_hi = _dequant_pair(uob_ref, uos_ref, sel16)

    xe = xe_ref[...]
    xo = xo_ref[...]
    gate_e = _dot_nt(xe, wge_lo) + _dot_nt(xo, wge_hi) + gbe_ref[0]
    gate_o = _dot_nt(xe, wgo_lo) + _dot_nt(xo, wgo_hi) + gbo_ref[0]
    up_e = _dot_nt(xe, wue_lo) + _dot_nt(xo, wue_hi) + ube_ref[0]
    up_o = _dot_nt(xe, wuo_lo) + _dot_nt(xo, wuo_hi) + ubo_ref[0]

    def _act(gate, up):
        gate = jnp.minimum(gate, LIMIT)
        up = jnp.clip(up, -LIMIT, LIMIT)
        glu = gate * (1.0 / (1.0 + jnp.exp(-ALPHA * gate)))
        return ((up + 1.0) * glu).astype(jnp.bfloat16)

    act_e = _act(gate_e, up_e)
    act_o = _act(gate_o, up_o)

    db = lax.broadcasted_iota(jnp.int32, (dns_ref.shape[2], FT // 2), 0)
    dc = lax.broadcasted_iota(jnp.int32, (dns_ref.shape[2], FT // 2), 1) // 16
    selj = (db == j * (FT // 32) + dc).astype(jnp.bfloat16)
    dsc = jnp.dot(dns_ref[0].astype(jnp.bfloat16), selj,
                  preferred_element_type=jnp.float32)

    dbytes = dnb_ref[0].astype(jnp.int32)
    wd_lo = (_nib2val(dbytes & 15) * dsc).astype(jnp.bfloat16)
    wd_hi = (_nib2val(dbytes >> 4) * dsc).astype(jnp.bfloat16)

    down = _dot_nt(act_e, wd_lo) + _dot_nt(act_o, wd_hi)

    wts = wts_ref[...]
    ei = lax.broadcasted_iota(jnp.int32, wts.shape, 1)
    w_col = jnp.sum(wts * (ei == e).astype(jnp.float32), axis=1,
                    keepdims=True)

    bias_gate = jnp.where(j == 0, 1.0, 0.0)
    contrib = w_col * (down + bias_gate * dnbias_ref[0])

    @pl.when(jnp.logical_and(e == 0, j == 0))
    def _():
        out_ref[...] = contrib

    @pl.when(jnp.logical_or(e != 0, j != 0))
    def _():
        out_ref[...] += contrib


@jax.jit
def kernel(x, router_w, router_b, gu_blocks, gu_scales, gu_bias, dn_blocks,
           dn_scales, dn_bias):
    Bb, Tt, H = x.shape
    E, FF2 = gu_bias.shape
    FF = FF2 // 2
    T = Bb * Tt
    J = FF // FT
    M = FF // 2
    MT = FT // 2

    xf = x.reshape(T, H)
    xe = xf[:, 0::2].astype(jnp.bfloat16)
    xo = xf[:, 1::2].astype(jnp.bfloat16)

    gu5 = gu_blocks.reshape(E, M, 2, 2, H // 2)
    gusc5 = jnp.exp2(gu_scales.astype(jnp.float32) - 127.0) \
        .reshape(E, M, 2, 2, H // 32)
    geb, gob = gu5[:, :, 0, 0, :], gu5[:, :, 1, 0, :]
    ueb, uob = gu5[:, :, 0, 1, :], gu5[:, :, 1, 1, :]
    ges, gos = gusc5[:, :, 0, 0, :], gusc5[:, :, 1, 0, :]
    ues, uos = gusc5[:, :, 0, 1, :], gusc5[:, :, 1, 1, :]
    gbe = gu_bias[:, 0::4].reshape(E, 1, M)
    gbo = gu_bias[:, 2::4].reshape(E, 1, M)
    ube = gu_bias[:, 1::4].reshape(E, 1, M)
    ubo = gu_bias[:, 3::4].reshape(E, 1, M)

    dnb = dn_blocks.reshape(E, H, FF // 2)
    dns = jnp.exp2(dn_scales.astype(jnp.float32) - 127.0)
    dnbias = dn_bias.reshape(E, 1, H)

    wts = pl.pallas_call(
        _router_kernel,
        out_shape=jax.ShapeDtypeStruct((T, E), jnp.float32),
    )(xf, router_w, router_b.reshape(1, E))

    wspec = pl.BlockSpec((1, MT, H // 2), lambda e, j: (e, j, 0))
    sspec = pl.BlockSpec((1, MT, H // 32), lambda e, j: (e, j, 0))
    bspec = pl.BlockSpec((1, 1, MT), lambda e, j: (e, 0, j))

    out = pl.pallas_call(
        _moe_kernel,
        grid=(E, J),
        in_specs=[
            pl.BlockSpec((T, H // 2), lambda e, j: (0, 0)),
            pl.BlockSpec((T, H // 2), lambda e, j: (0, 0)),
            pl.BlockSpec((T, E), lambda e, j: (0, 0)),
            wspec, wspec, wspec, wspec,
            sspec, sspec, sspec, sspec,
            bspec, bspec, bspec, bspec,
            pl.BlockSpec((1, H, FT // 2), lambda e, j: (e, 0, j)),
            pl.BlockSpec((1, H, FF // 32), lambda e, j: (e, 0, 0)),
            pl.BlockSpec((1, 1, H), lambda e, j: (e, 0, 0)),
        ],
        out_specs=pl.BlockSpec((T, H), lambda e, j: (0, 0)),
        out_shape=jax.ShapeDtypeStruct((T, H), jnp.float32),
    )(xe, xo, wts, geb, gob, ueb, uob, ges, gos, ues, uos,
      gbe, gbo, ube, ubo, dnb, dns, dnbias)

    return out.reshape(Bb, Tt, H)


# speedup vs baseline: 1.6029x; 1.6029x over previous
import numpy as np
import jax
import jax.numpy as jnp
from jax import lax
from jax.experimental import pallas as pl
from jax.experimental.pallas import tpu as pltpu

ALPHA = 1.702
LIMIT = 7.0
FT = 512
TILE = 256


def _nib2val(n):
    m = n & 7
    mag = jnp.where(
        m == 0, 0.0,
        jnp.where(m == 1, 0.5,
        jnp.where(m == 2, 1.0,
        jnp.where(m == 3, 1.5,
        jnp.where(m == 4, 2.0,
        jnp.where(m == 5, 3.0,
        jnp.where(m == 6, 4.0, 6.0)))))))
    return jnp.where(n >= 8, -mag, mag)


def _dot_nt(a, b):
    return lax.dot_general(a, b, (((1,), (1,)), ((), ())),
                           preferred_element_type=jnp.float32)


def _router_kernel(x_ref, rw_ref, rb_ref, i1_ref, i2_ref, w1_ref, w2_ref):
    x = x_ref[...]
    logits = _dot_nt(x, rw_ref[...]) + rb_ref[...]
    ei = lax.broadcasted_iota(jnp.int32, logits.shape, 1)
    m1 = jnp.max(logits, axis=1, keepdims=True)
    is1 = logits == m1
    masked = jnp.where(is1, -jnp.inf, logits)
    m2 = jnp.max(masked, axis=1, keepdims=True)
    is2 = masked == m2
    p1 = 1.0 / (1.0 + jnp.exp(m2 - m1))
    i1_ref[...] = jnp.sum(jnp.where(is1, ei, 0), axis=1, keepdims=True)
    i2_ref[...] = jnp.sum(jnp.where(is2, ei, 0), axis=1, keepdims=True)
    w1_ref[...] = p1
    w2_ref[...] = 1.0 - p1


def _moe_kernel(te_ref, tokc_ref, tokr_ref, wc_ref,
                xe_ref, xo_ref, gug_ref, guu_ref, gugs_ref, guus_ref,
                gb_ref, ub_ref, dnb_ref, dns_ref, dnbias_ref, out_ref,
                wgl_ref, wgh_ref, wul_ref, wuh_ref, wdl_ref, wdh_ref):
    i = pl.program_id(0)
    T = xe_ref.shape[0]
    FF = gug_ref.shape[1]
    nsc = gugs_ref.shape[2]
    ndc = dnb_ref.shape[2]

    prev = te_ref[jnp.maximum(i - 1, 0)]
    changed = jnp.logical_or(i == 0, te_ref[i] != prev)

    @pl.when(changed)
    def _dequant():
        ci = lax.broadcasted_iota(jnp.int32, (nsc, 16 * nsc), 1) // 16
        bi = lax.broadcasted_iota(jnp.int32, (nsc, 16 * nsc), 0)
        sel16 = (ci == bi).astype(jnp.bfloat16)
        gsc = jnp.dot(gugs_ref[0].astype(jnp.bfloat16), sel16,
                      preferred_element_type=jnp.float32)
        usc = jnp.dot(guus_ref[0].astype(jnp.bfloat16), sel16,
                      preferred_element_type=jnp.float32)
        gbytes = gug_ref[0].astype(jnp.int32)
        ubytes = guu_ref[0].astype(jnp.int32)
        wgl_ref[...] = (_nib2val(gbytes & 15) * gsc).astype(jnp.bfloat16)
        wgh_ref[...] = (_nib2val(gbytes >> 4) * gsc).astype(jnp.bfloat16)
        wul_ref[...] = (_nib2val(ubytes & 15) * usc).astype(jnp.bfloat16)
        wuh_ref[...] = (_nib2val(ubytes >> 4) * usc).astype(jnp.bfloat16)

        di = lax.broadcasted_iota(jnp.int32, (dns_ref.shape[2], ndc), 1) // 16
        dbi = lax.broadcasted_iota(jnp.int32, (dns_ref.shape[2], ndc), 0)
        seld = (di == dbi).astype(jnp.bfloat16)
        dsc = jnp.dot(dns_ref[0].astype(jnp.bfloat16), seld,
                      preferred_element_type=jnp.float32)
        dbytes = dnb_ref[0].astype(jnp.int32)
        wdl_ref[...] = (_nib2val(dbytes & 15) * dsc).astype(jnp.bfloat16)
        wdh_ref[...] = (_nib2val(dbytes >> 4) * dsc).astype(jnp.bfloat16)

    tok_col = tokc_ref[0]
    oh = (lax.broadcasted_iota(jnp.int32, (TILE, T), 1)
          == tok_col).astype(jnp.bfloat16)
    xg_e = jnp.dot(oh, xe_ref[...],
                   preferred_element_type=jnp.float32).astype(jnp.bfloat16)
    xg_o = jnp.dot(oh, xo_ref[...],
                   preferred_element_type=jnp.float32).astype(jnp.bfloat16)

    gate = _dot_nt(xg_e, wgl_ref[...]) + _dot_nt(xg_o, wgh_ref[...]) \
        + gb_ref[0]
    up = _dot_nt(xg_e, wul_ref[...]) + _dot_nt(xg_o, wuh_ref[...]) \
        + ub_ref[0]

    gate = jnp.minimum(gate, LIMIT)
    up = jnp.clip(up, -LIMIT, LIMIT)
    glu = gate * (1.0 / (1.0 + jnp.exp(-ALPHA * gate)))
    act = ((up + 1.0) * glu).astype(jnp.bfloat16)

    down = jnp.zeros((TILE, wdl_ref.shape[0]), jnp.float32)
    half = FT // 2
    for j in range(FF // FT):
        a_e = act[:, j * FT:j * FT + half]
        a_o = act[:, j * FT + half:(j + 1) * FT]
        down += _dot_nt(a_e, wdl_ref[:, j * half:(j + 1) * half])
        down += _dot_nt(a_o, wdh_ref[:, j * half:(j + 1) * half])

    down_w = ((down + dnbias_ref[0]) * wc_ref[0]).astype(jnp.bfloat16)

    tok_row = tokr_ref[0]
    oht = (lax.broadcasted_iota(jnp.int32, (T, TILE), 0)
           == tok_row).astype(jnp.bfloat16)
    contrib = jnp.dot(oht, down_w, preferred_element_type=jnp.float32)

    @pl.when(i == 0)
    def _():
        out_ref[...] = contrib

    @pl.when(i != 0)
    def _():
        out_ref[...] += contrib


@jax.jit
def kernel(x, router_w, router_b, gu_blocks, gu_scales, gu_bias, dn_blocks,
           dn_scales, dn_bias):
    Bb, Tt, H = x.shape
    E, FF2 = gu_bias.shape
    FF = FF2 // 2
    T = Bb * Tt
    NPAD = 2 * T + E * TILE
    NT = NPAD // TILE

    xf = x.reshape(T, H)
    xe = xf[:, 0::2].astype(jnp.bfloat16)
    xo = xf[:, 1::2].astype(jnp.bfloat16)

    idx = np.arange(FF)
    within = idx % FT
    base = (idx // FT) * FT
    P = base + np.where(within < FT // 2, 2 * within,
                        2 * (within - FT // 2) + 1)

    gu_b_flat = gu_blocks.reshape(E, 2 * FF, H // 2)
    gug = gu_b_flat[:, 2 * P, :]
    guu = gu_b_flat[:, 2 * P + 1, :]
    gu_s = jnp.exp2(gu_scales.astype(jnp.float32) - 127.0)
    gugs = gu_s[:, 2 * P, :]
    guus = gu_s[:, 2 * P + 1, :]
    gb = gu_bias[:, 2 * P].reshape(E, 1, FF)
    ub = gu_bias[:, 2 * P + 1].reshape(E, 1, FF)

    dnb = dn_blocks.reshape(E, H, FF // 2)
    dns = jnp.exp2(dn_scales.astype(jnp.float32) - 127.0)
    dnbias = dn_bias.reshape(E, 1, H)

    i1, i2, w1, w2 = pl.pallas_call(
        _router_kernel,
        out_shape=[jax.ShapeDtypeStruct((T, 1), jnp.int32),
                   jax.ShapeDtypeStruct((T, 1), jnp.int32),
                   jax.ShapeDtypeStruct((T, 1), jnp.float32),
                   jax.ShapeDtypeStruct((T, 1), jnp.float32)],
    )(xf, router_w, router_b.reshape(1, E))

    ids = jnp.concatenate([i1[:, 0], i2[:, 0]])
    tws = jnp.concatenate([w1[:, 0], w2[:, 0]])
    toks = jnp.concatenate([jnp.arange(T, dtype=jnp.int32)] * 2)
    order = jnp.argsort(ids)
    ids_s = ids[order]
    toks_s = toks[order]
    tws_s = tws[order]
    counts = jnp.sum((ids[None, :] == jnp.arange(E)[:, None]), axis=1)
    pc = ((counts + TILE - 1) // TILE) * TILE
    ps_full = jnp.concatenate([jnp.zeros((1,), pc.dtype), jnp.cumsum(pc)])
    starts = jnp.concatenate([jnp.zeros((1,), counts.dtype),
                              jnp.cumsum(counts)])
    rank = jnp.arange(2 * T) - starts[ids_s]
    pos = ps_full[ids_s] + rank
    row_token = jnp.zeros((NPAD,), jnp.int32).at[pos].set(toks_s)
    row_w = jnp.zeros((NPAD,), jnp.float32).at[pos].set(tws_s)
    tile_expert = jnp.clip(
        jnp.sum(jnp.arange(NT)[:, None] * TILE >= ps_full[None, 1:],
                axis=1), 0, E - 1).astype(jnp.int32)

    tokc = row_token.reshape(NT, TILE, 1)
    tokr = row_token.reshape(NT, 1, TILE)
    wc = row_w.reshape(NT, TILE, 1)

    grid_spec = pltpu.PrefetchScalarGridSpec(
        num_scalar_prefetch=1,
        grid=(NT,),
        in_specs=[
            pl.BlockSpec((1, TILE, 1), lambda i, te: (i, 0, 0)),
            pl.BlockSpec((1, 1, TILE), lambda i, te: (i, 0, 0)),
            pl.BlockSpec((1, TILE, 1), lambda i, te: (i, 0, 0)),
            pl.BlockSpec((T, H // 2), lambda i, te: (0, 0)),
            pl.BlockSpec((T, H // 2), lambda i, te: (0, 0)),
            pl.BlockSpec((1, FF, H // 2), lambda i, te: (te[i], 0, 0)),
            pl.BlockSpec((1, FF, H // 2), lambda i, te: (te[i], 0, 0)),
            pl.BlockSpec((1, FF, H // 32), lambda i, te: (te[i], 0, 0)),
            pl.BlockSpec((1, FF, H // 32), lambda i, te: (te[i], 0, 0)),
            pl.BlockSpec((1, 1, FF), lambda i, te: (te[i], 0, 0)),
            pl.BlockSpec((1, 1, FF), lambda i, te: (te[i], 0, 0)),
            pl.BlockSpec((1, H, FF // 2), lambda i, te: (te[i], 0, 0)),
            pl.BlockSpec((1, H, FF // 32), lambda i, te: (te[i], 0, 0)),
            pl.BlockSpec((1, 1, H), lambda i, te: (te[i], 0, 0)),
        ],
        out_specs=pl.BlockSpec((T, H), lambda i, te: (0, 0)),
        scratch_shapes=[
            pltpu.VMEM((FF, H // 2), jnp.bfloat16),
            pltpu.VMEM((FF, H // 2), jnp.bfloat16),
            pltpu.VMEM((FF, H // 2), jnp.bfloat16),
            pltpu.VMEM((FF, H // 2), jnp.bfloat16),
            pltpu.VMEM((H, FF // 2), jnp.bfloat16),
            pltpu.VMEM((H, FF // 2), jnp.bfloat16),
        ],
    )

    out = pl.pallas_call(
        _moe_kernel,
        grid_spec=grid_spec,
        out_shape=jax.ShapeDtypeStruct((T, H), jnp.float32),
    )(tile_expert, tokc, tokr, wc, xe, xo, gug, guu, gugs, guus,
      gb, ub, dnb, dns, dnbias)

    return out.reshape(Bb, Tt, H)
